# Initial kernel scaffold; baseline (speedup 1.0000x reference)
import jax
import jax.numpy as jnp
from jax.experimental import pallas as pl


def kernel(L_indices, L_values, x, theta):
    raise NotImplementedError("write your pallas kernel here")


# speedup vs baseline: 3.0278x; 3.0278x over previous
import jax
import jax.numpy as jnp
from jax import lax
from jax.experimental import pallas as pl
from jax.experimental.pallas import tpu as pltpu
from jax.experimental.pallas import tpu_sc as plsc

N = 10000
D = 128
H = D // 2
E = 320000
NC = 2
NS = 16
NW = NC * NS
EPT = E // NW
K = 80
C = EPT // K
RZ0 = 632
RZL = N - (NS - 1) * RZ0


def _zero_rows(buf, acc, base, nrows):
    for j in range(nrows // K):
        pltpu.sync_copy(buf, acc.at[pl.ds(base + j * K, K)])
    rem = nrows % K
    if rem:
        pltpu.sync_copy(
            buf.at[pl.ds(0, rem)], acc.at[pl.ds(base + (nrows // K) * K, rem)]
        )


def _sc_body(cols_hbm, rows_hbm, vals_hbm, x0_hbm, x1_hbm, part_hbm,
             cidx, ridx, vals_v, buf, acc, sem):
    c = lax.axis_index("c")
    s = lax.axis_index("s")
    tid = c * NS + s
    base = s * RZ0

    pltpu.sync_copy(cols_hbm.at[tid], cidx)
    pltpu.sync_copy(rows_hbm.at[tid], ridx)
    pltpu.sync_copy(vals_hbm.at[tid], vals_v)

    for h in range(2):
        def zero_buf(i, _):
            for j in range(H // 16):
                buf[i, pl.ds(j * 16, 16)] = jnp.zeros((16,), jnp.float32)
            return 0
        lax.fori_loop(0, K, zero_buf, 0)

        @pl.when(s < NS - 1)
        def _zero_main():
            _zero_rows(buf, acc, base, RZ0)

        @pl.when(s == NS - 1)
        def _zero_last():
            _zero_rows(buf, acc, base, RZL)

        plsc.subcore_barrier()

        def chunk(ci, _):
            xh_hbm = x0_hbm if h == 0 else x1_hbm
            pltpu.async_copy(xh_hbm.at[cidx.at[ci]], buf, sem).wait()

            def scale(g, _):
                vv = vals_v[ci, pl.ds(g * 16, 16)]
                for ee in range(16):
                    e = g * 16 + ee
                    v = vv[ee]
                    for j in range(H // 16):
                        sl = pl.ds(j * 16, 16)
                        buf[e, sl] = buf[e, sl] * v
                return 0
            lax.fori_loop(0, K // 16, scale, 0)

            pltpu.sync_copy(buf, acc.at[ridx.at[ci]], add=True)
            return 0
        lax.fori_loop(0, C, chunk, 0)

        plsc.subcore_barrier()

        @pl.when(s < NS - 1)
        def _write_main():
            pltpu.sync_copy(
                acc.at[pl.ds(base, RZ0)], part_hbm.at[c, h, pl.ds(base, RZ0)]
            )

        @pl.when(s == NS - 1)
        def _write_last():
            pltpu.sync_copy(
                acc.at[pl.ds(base, RZL)], part_hbm.at[c, h, pl.ds(base, RZL)]
            )

        if h == 0:
            plsc.subcore_barrier()


def _sc_partials(cols, rows, vals, x0, x1):
    mesh = plsc.VectorSubcoreMesh(
        core_axis_name="c", subcore_axis_name="s", num_cores=NC, num_subcores=NS
    )
    return pl.kernel(
        _sc_body,
        out_type=jax.ShapeDtypeStruct((NC, 2, N, H), jnp.float32),
        mesh=mesh,
        compiler_params=pltpu.CompilerParams(use_tc_tiling_on_sc=False),
        scratch_types=[
            pltpu.VMEM((C, K), jnp.int32),
            pltpu.VMEM((C, K), jnp.int32),
            pltpu.VMEM((C, K), jnp.float32),
            pltpu.VMEM((K, H), jnp.float32),
            pltpu.VMEM_SHARED((N, H), jnp.float32),
            pltpu.SemaphoreType.DMA,
        ],
    )(cols, rows, vals, x0, x1)


def _tc_body(p_ref, th_ref, o_ref):
    lx = jnp.concatenate(
        [p_ref[0, 0] + p_ref[1, 0], p_ref[0, 1] + p_ref[1, 1]], axis=-1
    )
    o_ref[...] = jnp.dot(lx, th_ref[...], preferred_element_type=jnp.float32)


def _tc_combine(part, theta):
    RB = 1000
    return pl.pallas_call(
        _tc_body,
        grid=(N // RB,),
        in_specs=[
            pl.BlockSpec((NC, 2, RB, H), lambda i: (0, 0, i, 0)),
            pl.BlockSpec((D, D), lambda i: (0, 0)),
        ],
        out_specs=pl.BlockSpec((RB, D), lambda i: (i, 0)),
        out_shape=jax.ShapeDtypeStruct((N, D), jnp.float32),
    )(part, theta)


def kernel(L_indices, L_values, x, theta):
    rows = L_indices[0].astype(jnp.int32).reshape(NW, C, K)
    cols = L_indices[1].astype(jnp.int32).reshape(NW, C, K)
    vals = L_values.astype(jnp.float32).reshape(NW, C, K)
    x0 = x[:, :H]
    x1 = x[:, H:]
    part = _sc_partials(cols, rows, vals, x0, x1)
    return _tc_combine(part, theta)


# speedup vs baseline: 4.2945x; 1.4183x over previous
import jax
import jax.numpy as jnp
from jax import lax
from jax.experimental import pallas as pl
from jax.experimental.pallas import tpu as pltpu
from jax.experimental.pallas import tpu_sc as plsc

N = 10000
D = 128
H = D // 2
E = 320000
NC = 2
NS = 16
NW = NC * NS
EPT = E // NW
K = 80
C = EPT // K
RZ0 = 632
RZL = N - (NS - 1) * RZ0


def _zero_rows(buf, acc, base, nrows):
    for j in range(nrows // K):
        pltpu.sync_copy(buf, acc.at[pl.ds(base + j * K, K)])
    rem = nrows % K
    if rem:
        pltpu.sync_copy(
            buf.at[pl.ds(0, rem)], acc.at[pl.ds(base + (nrows // K) * K, rem)]
        )


def _sc_body(cols_hbm, rows_hbm, vals_hbm, x0_hbm, x1_hbm, part_hbm,
             cidx, ridx, vals_v, buf, buf1, acc, sem, sem1):
    c = lax.axis_index("c")
    s = lax.axis_index("s")
    tid = c * NS + s
    base = s * RZ0

    pltpu.sync_copy(cols_hbm.at[tid], cidx)
    pltpu.sync_copy(rows_hbm.at[tid], ridx)
    pltpu.sync_copy(vals_hbm.at[tid], vals_v)

    for h in range(2):
        def zero_buf(i, _):
            for j in range(H // 16):
                buf[i, pl.ds(j * 16, 16)] = jnp.zeros((16,), jnp.float32)
            return 0
        lax.fori_loop(0, K, zero_buf, 0)

        @pl.when(s < NS - 1)
        def _zero_main():
            _zero_rows(buf, acc, base, RZ0)

        @pl.when(s == NS - 1)
        def _zero_last():
            _zero_rows(buf, acc, base, RZL)

        plsc.subcore_barrier()

        xh_hbm = x0_hbm if h == 0 else x1_hbm

        def scale(ci, b):
            def scale_g(g, _):
                vv = vals_v[ci, pl.ds(g * 16, 16)]
                for ee in range(16):
                    e = g * 16 + ee
                    v = vv[ee]
                    for j in range(H // 16):
                        sl = pl.ds(j * 16, 16)
                        b[e, sl] = b[e, sl] * v
                return 0
            lax.fori_loop(0, K // 16, scale_g, 0)

        pltpu.async_copy(xh_hbm.at[cidx.at[0]], buf, sem)

        def pair(i, _):
            c0 = 2 * i
            pltpu.async_copy(xh_hbm.at[cidx.at[c0 + 1]], buf1, sem1)
            pltpu.make_async_copy(xh_hbm.at[cidx.at[c0]], buf, sem).wait()
            scale(c0, buf)
            pltpu.sync_copy(buf, acc.at[ridx.at[c0]], add=True)

            @pl.when(c0 + 2 < C)
            def _prefetch_next():
                pltpu.async_copy(xh_hbm.at[cidx.at[c0 + 2]], buf, sem)

            pltpu.make_async_copy(xh_hbm.at[cidx.at[c0 + 1]], buf1, sem1).wait()
            scale(c0 + 1, buf1)
            pltpu.sync_copy(buf1, acc.at[ridx.at[c0 + 1]], add=True)
            return 0
        lax.fori_loop(0, C // 2, pair, 0)

        pltpu.make_async_copy(xh_hbm.at[cidx.at[C - 1]], buf, sem).wait()
        scale(C - 1, buf)
        pltpu.sync_copy(buf, acc.at[ridx.at[C - 1]], add=True)

        plsc.subcore_barrier()

        @pl.when(s < NS - 1)
        def _write_main():
            pltpu.sync_copy(
                acc.at[pl.ds(base, RZ0)], part_hbm.at[c, h, pl.ds(base, RZ0)]
            )

        @pl.when(s == NS - 1)
        def _write_last():
            pltpu.sync_copy(
                acc.at[pl.ds(base, RZL)], part_hbm.at[c, h, pl.ds(base, RZL)]
            )

        if h == 0:
            plsc.subcore_barrier()


def _sc_partials(cols, rows, vals, x0, x1):
    mesh = plsc.VectorSubcoreMesh(
        core_axis_name="c", subcore_axis_name="s", num_cores=NC, num_subcores=NS
    )
    return pl.kernel(
        _sc_body,
        out_type=jax.ShapeDtypeStruct((NC, 2, N, H), jnp.float32),
        mesh=mesh,
        compiler_params=pltpu.CompilerParams(use_tc_tiling_on_sc=False),
        scratch_types=[
            pltpu.VMEM((C, K), jnp.int32),
            pltpu.VMEM((C, K), jnp.int32),
            pltpu.VMEM((C, K), jnp.float32),
            pltpu.VMEM((K, H), jnp.float32),
            pltpu.VMEM((K, H), jnp.float32),
            pltpu.VMEM_SHARED((N, H), jnp.float32),
            pltpu.SemaphoreType.DMA,
            pltpu.SemaphoreType.DMA,
        ],
    )(cols, rows, vals, x0, x1)


def _tc_body(p_ref, th_ref, o_ref):
    lx = jnp.concatenate(
        [p_ref[0, 0] + p_ref[1, 0], p_ref[0, 1] + p_ref[1, 1]], axis=-1
    )
    o_ref[...] = jnp.dot(lx, th_ref[...], preferred_element_type=jnp.float32)


def _tc_combine(part, theta):
    RB = 1000
    return pl.pallas_call(
        _tc_body,
        grid=(N // RB,),
        in_specs=[
            pl.BlockSpec((NC, 2, RB, H), lambda i: (0, 0, i, 0)),
            pl.BlockSpec((D, D), lambda i: (0, 0)),
        ],
        out_specs=pl.BlockSpec((RB, D), lambda i: (i, 0)),
        out_shape=jax.ShapeDtypeStruct((N, D), jnp.float32),
    )(part, theta)


def kernel(L_indices, L_values, x, theta):
    rows = L_indices[0].astype(jnp.int32).reshape(NW, C, K)
    cols = L_indices[1].astype(jnp.int32).reshape(NW, C, K)
    vals = L_values.astype(jnp.float32).reshape(NW, C, K)
    x0 = x[:, :H]
    x1 = x[:, H:]
    part = _sc_partials(cols, rows, vals, x0, x1)
    return _tc_combine(part, theta)


# speedup vs baseline: 9.0275x; 2.1021x over previous
import jax
import jax.numpy as jnp
from jax import lax
from jax.experimental import pallas as pl
from jax.experimental.pallas import tpu as pltpu
from jax.experimental.pallas import tpu_sc as plsc

N = 10000
D = 128
H = D // 2
E = 320000
NC = 2
NS = 16
NW = NC * NS
EPT = E // NW
K = 80
C = EPT // K
RZ0 = 632
RZL = N - (NS - 1) * RZ0


def _zero_rows(buf, acc, base, nrows):
    for j in range(nrows // K):
        pltpu.sync_copy(buf, acc.at[pl.ds(base + j * K, K)])
    rem = nrows % K
    if rem:
        pltpu.sync_copy(
            buf.at[pl.ds(0, rem)], acc.at[pl.ds(base + (nrows // K) * K, rem)]
        )


def _sc_body(cols_hbm, rows_hbm, vals_hbm, x0_hbm, x1_hbm, part_hbm,
             cidx, ridx, vals_v, gbuf0, gbuf1, sbuf0, sbuf1, acc,
             gsem0, gsem1, ssem0, ssem1):
    c = lax.axis_index("c")
    s = lax.axis_index("s")
    tid = c * NS + s
    base = s * RZ0

    pltpu.sync_copy(cols_hbm.at[tid], cidx)
    pltpu.sync_copy(rows_hbm.at[tid], ridx)
    pltpu.sync_copy(vals_hbm.at[tid], vals_v)

    for h in range(2):
        def zero_buf(i, _):
            for j in range(H // 16):
                gbuf0[i, pl.ds(j * 16, 16)] = jnp.zeros((16,), jnp.float32)
            return 0
        lax.fori_loop(0, K, zero_buf, 0)

        @pl.when(s < NS - 1)
        def _zero_main():
            _zero_rows(gbuf0, acc, base, RZ0)

        @pl.when(s == NS - 1)
        def _zero_last():
            _zero_rows(gbuf0, acc, base, RZL)

        plsc.subcore_barrier()

        xh_hbm = x0_hbm if h == 0 else x1_hbm

        def scale(ci, gb, sb):
            def scale_g(g, _):
                vv = vals_v[ci, pl.ds(g * 16, 16)]
                for ee in range(16):
                    e = g * 16 + ee
                    v = vv[ee]
                    for j in range(H // 16):
                        sl = pl.ds(j * 16, 16)
                        sb[e, sl] = gb[e, sl] * v
                return 0
            lax.fori_loop(0, K // 16, scale_g, 0)

        def do_chunk(ci, i, gb, sb, gsem, ssem):
            pltpu.make_async_copy(xh_hbm.at[cidx.at[ci]], gb, gsem).wait()

            @pl.when(i > 0)
            def _wait_prev_scatter():
                pltpu.make_async_copy(sb, acc.at[ridx.at[ci]], ssem).wait()

            scale(ci, gb, sb)

            @pl.when(ci + 2 < C)
            def _prefetch_next():
                pltpu.async_copy(xh_hbm.at[cidx.at[ci + 2]], gb, gsem)

            pltpu.async_copy(sb, acc.at[ridx.at[ci]], ssem, add=True)

        pltpu.async_copy(xh_hbm.at[cidx.at[0]], gbuf0, gsem0)
        pltpu.async_copy(xh_hbm.at[cidx.at[1]], gbuf1, gsem1)

        def pair(i, _):
            do_chunk(2 * i, i, gbuf0, sbuf0, gsem0, ssem0)
            do_chunk(2 * i + 1, i, gbuf1, sbuf1, gsem1, ssem1)
            return 0
        lax.fori_loop(0, C // 2, pair, 0)

        pltpu.make_async_copy(xh_hbm.at[cidx.at[C - 1]], gbuf0, gsem0).wait()
        pltpu.make_async_copy(sbuf0, acc.at[ridx.at[C - 1]], ssem0).wait()
        scale(C - 1, gbuf0, sbuf0)
        pltpu.async_copy(sbuf0, acc.at[ridx.at[C - 1]], ssem0, add=True)
        pltpu.make_async_copy(sbuf0, acc.at[ridx.at[C - 1]], ssem0).wait()
        pltpu.make_async_copy(sbuf1, acc.at[ridx.at[C - 1]], ssem1).wait()

        plsc.subcore_barrier()

        @pl.when(s < NS - 1)
        def _write_main():
            pltpu.sync_copy(
                acc.at[pl.ds(base, RZ0)], part_hbm.at[c, h, pl.ds(base, RZ0)]
            )

        @pl.when(s == NS - 1)
        def _write_last():
            pltpu.sync_copy(
                acc.at[pl.ds(base, RZL)], part_hbm.at[c, h, pl.ds(base, RZL)]
            )

        if h == 0:
            plsc.subcore_barrier()


def _sc_partials(cols, rows, vals, x0, x1):
    mesh = plsc.VectorSubcoreMesh(
        core_axis_name="c", subcore_axis_name="s", num_cores=NC, num_subcores=NS
    )
    return pl.kernel(
        _sc_body,
        out_type=jax.ShapeDtypeStruct((NC, 2, N, H), jnp.float32),
        mesh=mesh,
        compiler_params=pltpu.CompilerParams(use_tc_tiling_on_sc=False),
        scratch_types=[
            pltpu.VMEM((C, K), jnp.int32),
            pltpu.VMEM((C, K), jnp.int32),
            pltpu.VMEM((C, K), jnp.float32),
            pltpu.VMEM((K, H), jnp.float32),
            pltpu.VMEM((K, H), jnp.float32),
            pltpu.VMEM((K, H), jnp.float32),
            pltpu.VMEM((K, H), jnp.float32),
            pltpu.VMEM_SHARED((N, H), jnp.float32),
            pltpu.SemaphoreType.DMA,
            pltpu.SemaphoreType.DMA,
            pltpu.SemaphoreType.DMA,
            pltpu.SemaphoreType.DMA,
        ],
    )(cols, rows, vals, x0, x1)


def _tc_body(p_ref, th_ref, o_ref):
    lx = jnp.concatenate(
        [p_ref[0, 0] + p_ref[1, 0], p_ref[0, 1] + p_ref[1, 1]], axis=-1
    )
    o_ref[...] = jnp.dot(lx, th_ref[...], preferred_element_type=jnp.float32)


def _tc_combine(part, theta):
    RB = 1000
    return pl.pallas_call(
        _tc_body,
        grid=(N // RB,),
        in_specs=[
            pl.BlockSpec((NC, 2, RB, H), lambda i: (0, 0, i, 0)),
            pl.BlockSpec((D, D), lambda i: (0, 0)),
        ],
        out_specs=pl.BlockSpec((RB, D), lambda i: (i, 0)),
        out_shape=jax.ShapeDtypeStruct((N, D), jnp.float32),
    )(part, theta)


def kernel(L_indices, L_values, x, theta):
    rows = L_indices[0].astype(jnp.int32).reshape(NW, C, K)
    cols = L_indices[1].astype(jnp.int32).reshape(NW, C, K)
    vals = L_values.astype(jnp.float32).reshape(NW, C, K)
    x0 = x[:, :H]
    x1 = x[:, H:]
    part = _sc_partials(cols, rows, vals, x0, x1)
    return _tc_combine(part, theta)
